# baseline (speedup 1.0000x reference)
import jax
import jax.numpy as jnp
from jax.experimental import pallas as pl
from jax.experimental.pallas import tpu as pltpu
from jax.experimental.pallas import tpu_sc as plsc

TOPK = 2
BLK = 128
GW = 128
SUBW = 256
RSUB = 4


def _sc_mesh():
    return plsc.VectorSubcoreMesh(core_axis_name="c", subcore_axis_name="s")


def _gather_rows32(data, idx):
    M = idx.shape[1]
    W = data.shape[1]

    @pl.kernel(out_type=jax.ShapeDtypeStruct((M, W), data.dtype), mesh=_sc_mesh())
    def gather_kernel(x_hbm, i_hbm, o_hbm):
        def body(i_vmem, o_vmem):
            pltpu.sync_copy(x_hbm.at[i_vmem.at[0]], o_vmem)

        pltpu.emit_pipeline(
            body,
            grid=(M // GW,),
            in_specs=[pl.BlockSpec((1, GW), lambda i: (0, i))],
            out_specs=[pl.BlockSpec((GW, W), lambda i: (i, 0))],
            core_axis_name=("c", "s"),
            dimension_semantics=(pltpu.PARALLEL,),
        )(i_hbm, o_hbm)

    return gather_kernel(data, idx)


def _scatter_rows32(data, idx_out, n_out):
    M = idx_out.shape[1]
    W = data.shape[1]

    n_src = data.shape[0] // GW

    @pl.kernel(out_type=jax.ShapeDtypeStruct((n_out, W), data.dtype),
               mesh=_sc_mesh())
    def scatter_kernel(x_hbm, io_hbm, o_hbm):
        def body(x_vmem, io_vmem):
            pltpu.sync_copy(x_vmem, o_hbm.at[io_vmem.at[0]])

        pltpu.emit_pipeline(
            body,
            grid=(M // GW,),
            in_specs=[pl.BlockSpec((GW, W), lambda i: (i % n_src, 0)),
                      pl.BlockSpec((1, GW), lambda i: (0, i))],
            out_specs=[],
            core_axis_name=("c", "s"),
            dimension_semantics=(pltpu.PARALLEL,),
        )(x_hbm, io_hbm)

    return scatter_kernel(data, idx_out)


def _ffn_body(be_ref, act_ref, x_ref, wg_ref, wu_ref, wd_ref, y_ref):
    b = pl.program_id(0)

    @pl.when(act_ref[b] > 0)
    def _():
        xs = x_ref[...]
        x = xs.reshape(-1, SUBW * RSUB)
        g = jax.lax.dot_general(x, wg_ref[0], (((1,), (1,)), ((), ())),
                                preferred_element_type=jnp.float32)
        u = jax.lax.dot_general(x, wu_ref[0], (((1,), (1,)), ((), ())),
                                preferred_element_type=jnp.float32)
        h = jax.nn.silu(g) * u
        y = jax.lax.dot_general(h, wd_ref[0], (((1,), (1,)), ((), ())),
                                preferred_element_type=jnp.float32)
        y_ref[...] = y.reshape(y_ref.shape)


def _combine_body(y0_ref, y1_ref, w0_ref, w1_ref, o_ref):
    w0 = w0_ref[0, 0, :]
    w1 = w1_ref[0, 0, :]
    v = y0_ref[...] * w0[:, None] + y1_ref[...] * w1[:, None]
    o_ref[...] = v.reshape(o_ref.shape)


def kernel(hidden_states, router_w, w_gate, w_up, w_down):
    B, S, D = hidden_states.shape
    E = router_w.shape[0]
    F = w_gate.shape[1]
    flat = hidden_states.reshape(-1, D)

    router_logits = flat @ router_w.T
    probs = jax.nn.softmax(router_logits, axis=-1)
    topv, topi = jax.lax.top_k(probs, TOPK)
    routing_weights = topv / jnp.sum(topv, axis=-1, keepdims=True)

    A = S * TOPK
    NB = (A + E * (BLK - 1)) // BLK
    NPAD = NB * BLK
    ids = topi.reshape(-1).astype(jnp.int32)
    onehot_i = (ids[:, None] == jnp.arange(E, dtype=jnp.int32)[None, :]
                ).astype(jnp.int32)
    rank = jnp.sum((jnp.cumsum(onehot_i, axis=0) - onehot_i) * onehot_i, axis=1)
    counts = jnp.sum(onehot_i, axis=0)
    padded = ((counts + BLK - 1) // BLK) * BLK
    offsets = jnp.concatenate(
        [jnp.zeros(1, jnp.int32), jnp.cumsum(padded)[:-1].astype(jnp.int32)])
    pos = jnp.sum(onehot_i * offsets[None, :], axis=1) + rank
    block_starts = jnp.arange(NB, dtype=jnp.int32) * BLK
    block_expert = (jnp.sum(
        (block_starts[:, None] >= offsets[None, :]).astype(jnp.int32), axis=1)
        - 1).astype(jnp.int32)
    total_padded = offsets[E - 1] + padded[E - 1]
    block_active = (block_starts < total_padded).astype(jnp.int32)

    R = D // SUBW
    flat32 = flat.reshape(S * R, SUBW)
    sub = jnp.arange(R, dtype=jnp.int32)[None, :]
    idx_out = jnp.concatenate(
        [(pos[k::TOPK][:, None] * R + sub).reshape(-1) for k in range(TOPK)]
    ).reshape(1, TOPK * S * R)
    x_sorted = _scatter_rows32(flat32, idx_out, NPAD * R)

    grid_spec = pltpu.PrefetchScalarGridSpec(
        num_scalar_prefetch=2,
        grid=(NB,),
        in_specs=[
            pl.BlockSpec((BLK * R, SUBW), lambda b, be, act: (b, 0)),
            pl.BlockSpec((1, F, D), lambda b, be, act: (be[b], 0, 0)),
            pl.BlockSpec((1, F, D), lambda b, be, act: (be[b], 0, 0)),
            pl.BlockSpec((1, D, F), lambda b, be, act: (be[b], 0, 0)),
        ],
        out_specs=pl.BlockSpec((BLK * R, SUBW), lambda b, be, act: (b, 0)),
    )
    y = pl.pallas_call(
        _ffn_body,
        grid_spec=grid_spec,
        out_shape=jax.ShapeDtypeStruct((NPAD * R, SUBW), jnp.float32),
        compiler_params=pltpu.CompilerParams(
            dimension_semantics=("parallel",),
            vmem_limit_bytes=100 * 1024 * 1024,
        ),
    )(block_expert, block_active, x_sorted, w_gate, w_up, w_down)

    pos_km = jnp.concatenate([pos[0::TOPK], pos[1::TOPK]])
    idx_cmb = (pos_km[:, None] * R + sub).reshape(1, A * R)
    yga = _gather_rows32(y, idx_cmb)

    SBR = 1024
    NSB = S * R // SBR
    w0 = jnp.repeat(routing_weights[:, 0], R).reshape(NSB, 1, SBR)
    w1 = jnp.repeat(routing_weights[:, 1], R).reshape(NSB, 1, SBR)
    final = pl.pallas_call(
        _combine_body,
        grid=(NSB,),
        in_specs=[
            pl.BlockSpec((SBR, SUBW), lambda i: (i, 0)),
            pl.BlockSpec((SBR, SUBW), lambda i: (i + NSB, 0)),
            pl.BlockSpec((1, 1, SBR), lambda i: (i, 0, 0)),
            pl.BlockSpec((1, 1, SBR), lambda i: (i, 0, 0)),
        ],
        out_specs=pl.BlockSpec((SBR // RSUB, D), lambda i: (i, 0)),
        out_shape=jax.ShapeDtypeStruct((S, D), jnp.float32),
    )(yga, yga, w0, w1)
    return final.reshape(B, S, D)


# speedup vs baseline: 1.5028x; 1.5028x over previous
import jax
import jax.numpy as jnp
from jax.experimental import pallas as pl
from jax.experimental.pallas import tpu as pltpu
from jax.experimental.pallas import tpu_sc as plsc

TOPK = 2
BLK = 512
GW = 128
SUBW = 256
RSUB = 4


def _sc_mesh():
    return plsc.VectorSubcoreMesh(core_axis_name="c", subcore_axis_name="s")


def _gather_rows32(data, idx):
    M = idx.shape[1]
    W = data.shape[1]

    @pl.kernel(out_type=jax.ShapeDtypeStruct((M, W), data.dtype), mesh=_sc_mesh())
    def gather_kernel(x_hbm, i_hbm, o_hbm):
        def body(i_vmem, o_vmem):
            pltpu.sync_copy(x_hbm.at[i_vmem.at[0]], o_vmem)

        pltpu.emit_pipeline(
            body,
            grid=(M // GW,),
            in_specs=[pl.BlockSpec((1, GW), lambda i: (0, i))],
            out_specs=[pl.BlockSpec((GW, W), lambda i: (i, 0))],
            core_axis_name=("c", "s"),
            dimension_semantics=(pltpu.PARALLEL,),
        )(i_hbm, o_hbm)

    return gather_kernel(data, idx)


def _scatter_rows32(data, idx_out, n_out):
    M = idx_out.shape[1]
    W = data.shape[1]

    n_src = data.shape[0] // GW

    @pl.kernel(out_type=jax.ShapeDtypeStruct((n_out, W), data.dtype),
               mesh=_sc_mesh())
    def scatter_kernel(x_hbm, io_hbm, o_hbm):
        def body(x_vmem, io_vmem):
            pltpu.sync_copy(x_vmem, o_hbm.at[io_vmem.at[0]])

        pltpu.emit_pipeline(
            body,
            grid=(M // GW,),
            in_specs=[pl.BlockSpec((GW, W), lambda i: (i % n_src, 0)),
                      pl.BlockSpec((1, GW), lambda i: (0, i))],
            out_specs=[],
            core_axis_name=("c", "s"),
            dimension_semantics=(pltpu.PARALLEL,),
        )(x_hbm, io_hbm)

    return scatter_kernel(data, idx_out)


def _ffn_body(be_ref, act_ref, x_ref, wg_ref, wu_ref, wd_ref, y_ref):
    b = pl.program_id(0)

    @pl.when(act_ref[b] > 0)
    def _():
        xs = x_ref[...]
        x = xs.reshape(-1, SUBW * RSUB)
        g = jax.lax.dot_general(x, wg_ref[0], (((1,), (1,)), ((), ())),
                                preferred_element_type=jnp.float32)
        u = jax.lax.dot_general(x, wu_ref[0], (((1,), (1,)), ((), ())),
                                preferred_element_type=jnp.float32)
        h = jax.nn.silu(g) * u
        y = jax.lax.dot_general(h, wd_ref[0], (((1,), (1,)), ((), ())),
                                preferred_element_type=jnp.float32)
        y_ref[...] = y.reshape(y_ref.shape)


def _combine_body(y0_ref, y1_ref, w0_ref, w1_ref, o_ref):
    w0 = w0_ref[0, 0, :]
    w1 = w1_ref[0, 0, :]
    v = y0_ref[...] * w0[:, None] + y1_ref[...] * w1[:, None]
    o_ref[...] = v.reshape(o_ref.shape)


def kernel(hidden_states, router_w, w_gate, w_up, w_down):
    B, S, D = hidden_states.shape
    E = router_w.shape[0]
    F = w_gate.shape[1]
    flat = hidden_states.reshape(-1, D)

    router_logits = flat @ router_w.T
    probs = jax.nn.softmax(router_logits, axis=-1)
    topv, topi = jax.lax.top_k(probs, TOPK)
    routing_weights = topv / jnp.sum(topv, axis=-1, keepdims=True)

    A = S * TOPK
    NB = (A + E * (BLK - 1)) // BLK
    NPAD = NB * BLK
    ids = topi.reshape(-1).astype(jnp.int32)
    onehot_i = (ids[:, None] == jnp.arange(E, dtype=jnp.int32)[None, :]
                ).astype(jnp.int32)
    rank = jnp.sum((jnp.cumsum(onehot_i, axis=0) - onehot_i) * onehot_i, axis=1)
    counts = jnp.sum(onehot_i, axis=0)
    padded = ((counts + BLK - 1) // BLK) * BLK
    offsets = jnp.concatenate(
        [jnp.zeros(1, jnp.int32), jnp.cumsum(padded)[:-1].astype(jnp.int32)])
    pos = jnp.sum(onehot_i * offsets[None, :], axis=1) + rank
    block_starts = jnp.arange(NB, dtype=jnp.int32) * BLK
    block_expert = (jnp.sum(
        (block_starts[:, None] >= offsets[None, :]).astype(jnp.int32), axis=1)
        - 1).astype(jnp.int32)
    total_padded = offsets[E - 1] + padded[E - 1]
    block_active = (block_starts < total_padded).astype(jnp.int32)

    R = D // SUBW
    flat32 = flat.reshape(S * R, SUBW)
    sub = jnp.arange(R, dtype=jnp.int32)[None, :]
    idx_out = jnp.concatenate(
        [(pos[k::TOPK][:, None] * R + sub).reshape(-1) for k in range(TOPK)]
    ).reshape(1, TOPK * S * R)
    x_sorted = _scatter_rows32(flat32, idx_out, NPAD * R)

    grid_spec = pltpu.PrefetchScalarGridSpec(
        num_scalar_prefetch=2,
        grid=(NB,),
        in_specs=[
            pl.BlockSpec((BLK * R, SUBW), lambda b, be, act: (b, 0)),
            pl.BlockSpec((1, F, D), lambda b, be, act: (be[b], 0, 0)),
            pl.BlockSpec((1, F, D), lambda b, be, act: (be[b], 0, 0)),
            pl.BlockSpec((1, D, F), lambda b, be, act: (be[b], 0, 0)),
        ],
        out_specs=pl.BlockSpec((BLK * R, SUBW), lambda b, be, act: (b, 0)),
    )
    y = pl.pallas_call(
        _ffn_body,
        grid_spec=grid_spec,
        out_shape=jax.ShapeDtypeStruct((NPAD * R, SUBW), jnp.float32),
        compiler_params=pltpu.CompilerParams(
            dimension_semantics=("parallel",),
            vmem_limit_bytes=100 * 1024 * 1024,
        ),
    )(block_expert, block_active, x_sorted, w_gate, w_up, w_down)

    pos_km = jnp.concatenate([pos[0::TOPK], pos[1::TOPK]])
    idx_cmb = (pos_km[:, None] * R + sub).reshape(1, A * R)
    yga = _gather_rows32(y, idx_cmb)

    SBR = 1024
    NSB = S * R // SBR
    w0 = jnp.repeat(routing_weights[:, 0], R).reshape(NSB, 1, SBR)
    w1 = jnp.repeat(routing_weights[:, 1], R).reshape(NSB, 1, SBR)
    final = pl.pallas_call(
        _combine_body,
        grid=(NSB,),
        in_specs=[
            pl.BlockSpec((SBR, SUBW), lambda i: (i, 0)),
            pl.BlockSpec((SBR, SUBW), lambda i: (i + NSB, 0)),
            pl.BlockSpec((1, 1, SBR), lambda i: (i, 0, 0)),
            pl.BlockSpec((1, 1, SBR), lambda i: (i, 0, 0)),
        ],
        out_specs=pl.BlockSpec((SBR // RSUB, D), lambda i: (i, 0)),
        out_shape=jax.ShapeDtypeStruct((S, D), jnp.float32),
    )(yga, yga, w0, w1)
    return final.reshape(B, S, D)
